# baseline (speedup 1.0000x reference)
import jax
import jax.numpy as jnp
from jax.experimental import pallas as pl
from jax.experimental.pallas import tpu as pltpu

N = 32768
D = 128
B = 8
L = 32
S = B * L
DELTA_V = 0.5
DELTA_D = 1.5
BN = 8192
NB = N // BN
XS = 64.0
CS = 448.0
F8 = jnp.float8_e4m3fn


def _body(x_ref, lab_ref, sb_ref, out_ref, sums_ref, g_ref, xn_ref):
    i = pl.program_id(0)

    @pl.when(i == 0)
    def _init():
        sums_ref[...] = jnp.zeros_like(sums_ref)

    x = x_ref[0]
    ss = jnp.sum(x * x, axis=1, keepdims=True)
    xn_bf = (x / (jnp.sqrt(ss) + 1e-8)).astype(jnp.bfloat16)
    xns_f8 = (xn_bf * XS).astype(F8)
    seg = sb_ref[i, :] * L + lab_ref[i, :]
    ids = jax.lax.broadcasted_iota(jnp.int32, (BN, S), 1)
    onehot = (seg[:, None] == ids).astype(F8)
    xn_ref[pl.ds(i * BN, BN), :] = xn_bf
    rhs = jnp.concatenate(
        [xns_f8, jnp.ones((BN, D), F8)], axis=1)
    sums_ref[...] += jax.lax.dot_general(
        onehot, rhs, (((0,), (0,)), ((), ())),
        preferred_element_type=jnp.float32)

    @pl.when(i == NB - 1)
    def _tail():
        counts = sums_ref[:, D:]
        safe = jnp.maximum(counts, 1.0)
        mus_s = sums_ref[:, :D] / safe
        mus = mus_s * (1.0 / XS)
        pres = (counts > 0.0).astype(jnp.float32)
        sb_i = jax.lax.broadcasted_iota(jnp.int32, (S, S), 0) // L
        sb_j = jax.lax.broadcasted_iota(jnp.int32, (S, S), 1) // L
        same_b = (sb_i == sb_j).astype(jnp.float32)
        m_rep = jax.lax.dot_general(
            same_b, pres, (((1,), (0,)), ((), ())),
            preferred_element_type=jnp.float32)
        valid = (m_rep > 1.0).astype(jnp.float32)
        coef_s = CS * valid / (jnp.maximum(m_rep, 1.0) * safe)
        g_ref[:, :D] = mus_s.astype(F8)
        g_ref[:, D:] = coef_s.astype(F8)

        acc_rep = jnp.zeros((BN, D), jnp.float32)
        for j in range(NB):
            seg_j = sb_ref[j, :] * L + lab_ref[j, :]
            oh_j = (seg_j[:, None] == ids).astype(F8)
            gathered = jax.lax.dot_general(
                oh_j, g_ref[...], (((1,), (0,)), ((), ())),
                preferred_element_type=jnp.float32)
            xnj = xn_ref[j * BN:(j + 1) * BN, :].astype(jnp.float32)
            dist = jnp.sum(jnp.abs(gathered[:, :D] - XS * xnj), axis=1)
            h = jnp.maximum(dist * (1.0 / XS) - DELTA_V, 0.0)
            h2 = h * h
            acc_rep += gathered[:, D:] * h2[:, None]
        acc = jnp.sum(acc_rep) * (1.0 / (D * CS))

        noteye = (jax.lax.broadcasted_iota(jnp.int32, (L, L, 1), 0) !=
                  jax.lax.broadcasted_iota(jnp.int32, (L, L, 1), 1)
                  ).astype(jnp.float32)
        total_push = jnp.zeros((), jnp.float32)
        b_eff = jnp.zeros((), jnp.float32)
        for b in range(B):
            mub = mus[b * L:(b + 1) * L, :]
            pb = pres[b * L:(b + 1) * L, :]
            diff = jnp.abs(mub[:, None, :] - mub[None, :, :])
            pd = jnp.sum(diff, axis=2, keepdims=True)
            hinge = jnp.maximum(2.0 * DELTA_D - pd, 0.0) * noteye
            mask3 = pb[:, None, :] * pb[None, :, :]
            psum = jnp.sum(mask3 * (hinge * hinge)) * (1.0 / D)
            m_b = jnp.sum(pb) * (1.0 / D)
            denom = jnp.maximum(m_b * (m_b - 1.0), 1.0)
            validb = (m_b > 1.0).astype(jnp.float32)
            total_push += psum / denom * validb
            b_eff += (m_b > 0.0).astype(jnp.float32)
        b_eff = jnp.maximum(b_eff, 1.0)
        out_ref[...] = jnp.reshape((acc + total_push) / b_eff, (1, 1))


def _run(x3, lab2, sb2, interpret=False):
    return pl.pallas_call(
        _body,
        grid=(NB,),
        in_specs=[
            pl.BlockSpec((1, BN, D), lambda i: (i, 0, 0)),
            pl.BlockSpec((NB, BN), lambda i: (0, 0)),
            pl.BlockSpec((NB, BN), lambda i: (0, 0)),
        ],
        out_specs=pl.BlockSpec((1, 1), lambda i: (0, 0)),
        out_shape=jax.ShapeDtypeStruct((1, 1), jnp.float32),
        scratch_shapes=[
            pltpu.VMEM((S, 2 * D), jnp.float32),
            pltpu.VMEM((S, 2 * D), F8),
            pltpu.VMEM((N, D), jnp.bfloat16),
        ],
        compiler_params=pltpu.CompilerParams(
            dimension_semantics=("arbitrary",)),
        interpret=interpret,
    )(x3, lab2, sb2)


def kernel(outputs, labels, subbatch_indices):
    x3 = outputs.reshape(NB, BN, D)
    lab2 = labels.astype(jnp.int32).reshape(NB, BN)
    sb2 = subbatch_indices.astype(jnp.int32).reshape(NB, BN)
    out = _run(x3, lab2, sb2)
    return out[0, 0]


# speedup vs baseline: 1.2791x; 1.2791x over previous
import jax
import jax.numpy as jnp
from jax.experimental import pallas as pl
from jax.experimental.pallas import tpu as pltpu

N = 32768
D = 128
B = 8
L = 32
S = B * L
DELTA_V = 0.5
DELTA_D = 1.5
BN = 8192
NB = N // BN
XS = 64.0
CS = 448.0
F8 = jnp.float8_e4m3fn


def _body(x_ref, lab_ref, sb_ref, out_ref, sums_ref, g_ref, oh_ref, xn_ref):
    i = pl.program_id(0)

    @pl.when(i == 0)
    def _init():
        sums_ref[...] = jnp.zeros_like(sums_ref)

    ones_dd = jnp.ones((D, D), jnp.bfloat16)
    x_bf = x_ref[0].astype(jnp.bfloat16)
    ss = jax.lax.dot_general(
        x_bf * x_bf, ones_dd, (((1,), (0,)), ((), ())),
        preferred_element_type=jnp.float32)
    xn_bf = x_bf * jax.lax.rsqrt(ss).astype(jnp.bfloat16)
    xns_f8 = (xn_bf * XS).astype(F8)
    seg = sb_ref[i, :] * L + lab_ref[i, :]
    ids = jax.lax.broadcasted_iota(jnp.int32, (BN, S), 1)
    onehot = (seg[:, None] == ids).astype(F8)
    oh_ref[pl.ds(i * BN, BN), :] = onehot
    xn_ref[pl.ds(i * BN, BN), :] = xn_bf
    rhs = jnp.concatenate(
        [xns_f8, jnp.ones((BN, D), F8)], axis=1)
    sums_ref[...] += jax.lax.dot_general(
        onehot, rhs, (((0,), (0,)), ((), ())),
        preferred_element_type=jnp.float32)

    @pl.when(i == NB - 1)
    def _tail():
        counts = sums_ref[:, D:]
        safe = jnp.maximum(counts, 1.0)
        mus_s = sums_ref[:, :D] / safe
        mus = mus_s * (1.0 / XS)
        pres = (counts > 0.0).astype(jnp.float32)
        sb_i = jax.lax.broadcasted_iota(jnp.int32, (S, S), 0) // L
        sb_j = jax.lax.broadcasted_iota(jnp.int32, (S, S), 1) // L
        same_b = (sb_i == sb_j).astype(jnp.float32)
        m_rep = jax.lax.dot_general(
            same_b, pres, (((1,), (0,)), ((), ())),
            preferred_element_type=jnp.float32)
        valid = (m_rep > 1.0).astype(jnp.float32)
        coef_s = CS * valid / (jnp.maximum(m_rep, 1.0) * safe)
        g_ref[:, :D] = mus_s.astype(F8)
        g_ref[:, D:] = coef_s.astype(F8)

        acc_rep = jnp.zeros((BN, D), jnp.float32)
        for j in range(NB):
            oh_j = oh_ref[j * BN:(j + 1) * BN, :]
            gathered = jax.lax.dot_general(
                oh_j, g_ref[...], (((1,), (0,)), ((), ())),
                preferred_element_type=jnp.float32)
            xnj = xn_ref[j * BN:(j + 1) * BN, :].astype(jnp.float32)
            dist = jnp.sum(jnp.abs(gathered[:, :D] - XS * xnj), axis=1)
            h = jnp.maximum(dist * (1.0 / XS) - DELTA_V, 0.0)
            h2 = h * h
            acc_rep += gathered[:, D:] * h2[:, None]
        acc = jnp.sum(acc_rep) * (1.0 / (D * CS))

        noteye = (jax.lax.broadcasted_iota(jnp.int32, (L, L, 1), 0) !=
                  jax.lax.broadcasted_iota(jnp.int32, (L, L, 1), 1)
                  ).astype(jnp.float32)
        total_push = jnp.zeros((), jnp.float32)
        b_eff = jnp.zeros((), jnp.float32)
        for b in range(B):
            mub = mus[b * L:(b + 1) * L, :]
            pb = pres[b * L:(b + 1) * L, :]
            diff = jnp.abs(mub[:, None, :] - mub[None, :, :])
            pd = jnp.sum(diff, axis=2, keepdims=True)
            hinge = jnp.maximum(2.0 * DELTA_D - pd, 0.0) * noteye
            mask3 = pb[:, None, :] * pb[None, :, :]
            psum = jnp.sum(mask3 * (hinge * hinge)) * (1.0 / D)
            m_b = jnp.sum(pb) * (1.0 / D)
            denom = jnp.maximum(m_b * (m_b - 1.0), 1.0)
            validb = (m_b > 1.0).astype(jnp.float32)
            total_push += psum / denom * validb
            b_eff += (m_b > 0.0).astype(jnp.float32)
        b_eff = jnp.maximum(b_eff, 1.0)
        out_ref[...] = jnp.reshape((acc + total_push) / b_eff, (1, 1))


def _run(x3, lab2, sb2, interpret=False):
    return pl.pallas_call(
        _body,
        grid=(NB,),
        in_specs=[
            pl.BlockSpec((1, BN, D), lambda i: (i, 0, 0)),
            pl.BlockSpec((NB, BN), lambda i: (0, 0)),
            pl.BlockSpec((NB, BN), lambda i: (0, 0)),
        ],
        out_specs=pl.BlockSpec((1, 1), lambda i: (0, 0)),
        out_shape=jax.ShapeDtypeStruct((1, 1), jnp.float32),
        scratch_shapes=[
            pltpu.VMEM((S, 2 * D), jnp.float32),
            pltpu.VMEM((S, 2 * D), F8),
            pltpu.VMEM((N, S), F8),
            pltpu.VMEM((N, D), jnp.bfloat16),
        ],
        compiler_params=pltpu.CompilerParams(
            dimension_semantics=("arbitrary",)),
        interpret=interpret,
    )(x3, lab2, sb2)


def kernel(outputs, labels, subbatch_indices):
    x3 = outputs.reshape(NB, BN, D)
    lab2 = labels.astype(jnp.int32).reshape(NB, BN)
    sb2 = subbatch_indices.astype(jnp.int32).reshape(NB, BN)
    out = _run(x3, lab2, sb2)
    return out[0, 0]
